# baseline (speedup 1.0000x reference)
import functools

import jax
import jax.numpy as jnp
from jax import lax
from jax.experimental import pallas as pl
from jax.experimental.pallas import tpu as pltpu
from jax.experimental.pallas import tpu_sc as plsc

N_NODES = 10000
N_EDGES = 320000
N_RELS = 100
D = 128
LANES = 16

NC = 2
NS = 16
NW = NC * NS
CH = 80
N_CHUNKS = 125
SLAB_CHUNKS = 128
SUP = 16
N_SUP = 8
E_PER_TILE = N_CHUNKS * CH
ROWS_PER_TILE = 624
LAST_TILE_ROWS = N_NODES - (NS - 1) * ROWS_PER_TILE


def _sc_agg_body(src_hbm, dst_hbm, typ_hbm, emb_hbm, rel_hbm, zagg_hbm, zdeg_hbm,
                 agg_out, deg_out,
                 idx_src, idx_dst, idx_typ, emb_v2, rel_v2, ones_v,
                 agg_sh, deg_sh, sem_g, sem_r, sem_s, sem_d):
    cid = lax.axis_index("c")
    sid = lax.axis_index("s")
    wid = cid * NS + sid

    row0 = sid * ROWS_PER_TILE

    @pl.when(sid < NS - 1)
    def _zero_agg():
        pltpu.sync_copy(zagg_hbm.at[pl.ds(row0, ROWS_PER_TILE)],
                        agg_sh.at[pl.ds(row0, ROWS_PER_TILE)])

    @pl.when(sid == NS - 1)
    def _zero_agg_last():
        pltpu.sync_copy(zagg_hbm.at[pl.ds((NS - 1) * ROWS_PER_TILE, LAST_TILE_ROWS)],
                        agg_sh.at[pl.ds((NS - 1) * ROWS_PER_TILE, LAST_TILE_ROWS)])

    @pl.when(sid == 0)
    def _zero_deg():
        pltpu.sync_copy(zdeg_hbm, deg_sh)

    for k in range(CH // LANES):
        ones_v[pl.ds(k * LANES, LANES)] = jnp.full((LANES,), 1.0, jnp.float32)

    plsc.subcore_barrier()

    def super_body(s, carry):
        c0 = s * SUP
        pltpu.sync_copy(src_hbm.at[wid, pl.ds(c0, SUP)], idx_src)
        pltpu.sync_copy(dst_hbm.at[wid, pl.ds(c0, SUP)], idx_dst)
        pltpu.sync_copy(typ_hbm.at[wid, pl.ds(c0, SUP)], idx_typ)
        n_inner = jnp.minimum(SUP, N_CHUNKS - c0)

        pltpu.async_copy(emb_hbm.at[idx_src.at[0]], emb_v2.at[0], sem_g)
        pltpu.async_copy(rel_hbm.at[idx_typ.at[0]], rel_v2.at[0], sem_r)

        def chunk_body(i, c1):
            p = lax.rem(i, 2)
            q = 1 - p
            pltpu.make_async_copy(emb_hbm.at[pl.ds(0, CH)],
                                  emb_v2.at[p], sem_g).wait()
            pltpu.make_async_copy(rel_hbm.at[pl.ds(0, CH)],
                                  rel_v2.at[p], sem_r).wait()

            @pl.when(i > 0)
            def _retire():
                pltpu.make_async_copy(emb_v2.at[q],
                                      agg_sh.at[pl.ds(0, CH)], sem_s).wait()
                pltpu.make_async_copy(ones_v,
                                      deg_sh.at[pl.ds(0, CH)], sem_d).wait()

            @pl.when(i + 1 < n_inner)
            def _prefetch():
                pltpu.async_copy(emb_hbm.at[idx_src.at[i + 1]],
                                 emb_v2.at[q], sem_g)
                pltpu.async_copy(rel_hbm.at[idx_typ.at[i + 1]],
                                 rel_v2.at[q], sem_r)

            def mul_body(r, c2):
                for k in range(D // LANES):
                    sl = pl.ds(k * LANES, LANES)
                    emb_v2[p, r, sl] = emb_v2[p, r, sl] * rel_v2[p, r, sl]
                return c2
            lax.fori_loop(0, CH, mul_body, 0)

            pltpu.async_copy(emb_v2.at[p], agg_sh.at[idx_dst.at[i]],
                             sem_s, add=True)
            pltpu.async_copy(ones_v, deg_sh.at[idx_dst.at[i]],
                             sem_d, add=True)
            return c1

        lax.fori_loop(0, n_inner, chunk_body, 0)

        pltpu.make_async_copy(emb_v2.at[lax.rem(n_inner - 1, 2)],
                              agg_sh.at[pl.ds(0, CH)], sem_s).wait()
        pltpu.make_async_copy(ones_v, deg_sh.at[pl.ds(0, CH)], sem_d).wait()
        return carry

    lax.fori_loop(0, N_SUP, super_body, 0)

    plsc.subcore_barrier()

    @pl.when(sid < NS - 1)
    def _write_agg():
        pltpu.sync_copy(agg_sh.at[pl.ds(row0, ROWS_PER_TILE)],
                        agg_out.at[cid, pl.ds(row0, ROWS_PER_TILE)])

    @pl.when(sid == NS - 1)
    def _write_agg_last():
        pltpu.sync_copy(agg_sh.at[pl.ds((NS - 1) * ROWS_PER_TILE, LAST_TILE_ROWS)],
                        agg_out.at[cid, pl.ds((NS - 1) * ROWS_PER_TILE, LAST_TILE_ROWS)])

    @pl.when(sid == 0)
    def _write_deg():
        pltpu.sync_copy(deg_sh, deg_out.at[cid])


_sc_agg = functools.partial(
    pl.kernel,
    out_type=[
        jax.ShapeDtypeStruct((NC, N_NODES, D), jnp.float32),
        jax.ShapeDtypeStruct((NC, N_NODES), jnp.float32),
    ],
    mesh=plsc.VectorSubcoreMesh(core_axis_name="c", subcore_axis_name="s"),
    scratch_types=[
        pltpu.VMEM((SUP, CH), jnp.int32),
        pltpu.VMEM((SUP, CH), jnp.int32),
        pltpu.VMEM((SUP, CH), jnp.int32),
        pltpu.VMEM((2, CH, D), jnp.float32),
        pltpu.VMEM((2, CH, D), jnp.float32),
        pltpu.VMEM((CH,), jnp.float32),
        pltpu.VMEM_SHARED((N_NODES, D), jnp.float32),
        pltpu.VMEM_SHARED((N_NODES,), jnp.float32),
        pltpu.SemaphoreType.DMA,
        pltpu.SemaphoreType.DMA,
        pltpu.SemaphoreType.DMA,
        pltpu.SemaphoreType.DMA,
    ],
)(_sc_agg_body)


ROW_BLK = 1000


def _tc_finish_body(aggp_ref, degp_ref, emb_ref, w_ref, wl_ref, b_ref, out_ref):
    agg = aggp_ref[0] + aggp_ref[1]
    deg = degp_ref[0, :, 0] + degp_ref[1, :, 0]
    norm = 1.0 / jnp.maximum(deg, 1.0)
    x = agg * norm[:, None]
    acc = jnp.dot(x, w_ref[...], preferred_element_type=jnp.float32)
    acc = acc + jnp.dot(emb_ref[...], wl_ref[...],
                        preferred_element_type=jnp.float32)
    out_ref[...] = jnp.tanh(acc + b_ref[...])


def _tc_finish(aggp, degp, emb, W, W_loop, b2d):
    grid = (N_NODES // ROW_BLK,)
    return pl.pallas_call(
        _tc_finish_body,
        grid=grid,
        in_specs=[
            pl.BlockSpec((NC, ROW_BLK, D), lambda i: (0, i, 0)),
            pl.BlockSpec((NC, ROW_BLK, 1), lambda i: (0, i, 0)),
            pl.BlockSpec((ROW_BLK, D), lambda i: (i, 0)),
            pl.BlockSpec((D, D), lambda i: (0, 0)),
            pl.BlockSpec((D, D), lambda i: (0, 0)),
            pl.BlockSpec((1, D), lambda i: (0, 0)),
        ],
        out_specs=pl.BlockSpec((ROW_BLK, D), lambda i: (i, 0)),
        out_shape=jax.ShapeDtypeStruct((N_NODES, D), jnp.float32),
    )(aggp, degp.reshape(NC, N_NODES, 1), emb, W, W_loop, b2d)


def _slab(x):
    x = x.reshape(NW, E_PER_TILE)
    x = jnp.pad(x, ((0, 0), (0, SLAB_CHUNKS * CH - E_PER_TILE)))
    return x.reshape(NW, SLAB_CHUNKS, CH)


def kernel(t, emb, edge_index, edge_type, W, W_loop, rel_emb, b):
    src = _slab(edge_index[0])
    dst = _slab(edge_index[1])
    typ = _slab(edge_type)
    zagg = jnp.zeros((N_NODES, D), jnp.float32)
    zdeg = jnp.zeros((N_NODES,), jnp.float32)
    aggp, degp = _sc_agg(src, dst, typ, emb, rel_emb, zagg, zdeg)
    return _tc_finish(aggp, degp, emb, W, W_loop, b.reshape(1, D))


# speedup vs baseline: 1.5781x; 1.5781x over previous
import functools

import jax
import jax.numpy as jnp
from jax import lax
from jax.experimental import pallas as pl
from jax.experimental.pallas import tpu as pltpu
from jax.experimental.pallas import tpu_sc as plsc

N_NODES = 10000
N_EDGES = 320000
N_RELS = 100
D = 128
LANES = 16

NC = 2
NS = 16
NW = NC * NS
CH = 80
N_CHUNKS = 125
SLAB_CHUNKS = 128
SUP = 16
N_SUP = 8
E_PER_TILE = N_CHUNKS * CH
ROWS_PER_TILE = 624
LAST_TILE_ROWS = N_NODES - (NS - 1) * ROWS_PER_TILE


def _sc_agg_body(src_hbm, dst_hbm, typ_hbm, emb_hbm, rel_hbm, zagg_hbm, zdeg_hbm,
                 agg_out, deg_out,
                 idx_src, idx_dst, idx_typ, emb_v2, rel_v2, ones_v,
                 agg_sh, deg_sh, sem_g, sem_r, sem_s, sem_d):
    cid = lax.axis_index("c")
    sid = lax.axis_index("s")
    wid = cid * NS + sid

    row0 = sid * ROWS_PER_TILE

    @pl.when(sid < NS - 1)
    def _zero_agg():
        pltpu.sync_copy(zagg_hbm.at[pl.ds(row0, ROWS_PER_TILE)],
                        agg_sh.at[pl.ds(row0, ROWS_PER_TILE)])

    @pl.when(sid == NS - 1)
    def _zero_agg_last():
        pltpu.sync_copy(zagg_hbm.at[pl.ds((NS - 1) * ROWS_PER_TILE, LAST_TILE_ROWS)],
                        agg_sh.at[pl.ds((NS - 1) * ROWS_PER_TILE, LAST_TILE_ROWS)])

    @pl.when(sid == 0)
    def _zero_deg():
        pltpu.sync_copy(zdeg_hbm, deg_sh)

    for k in range(CH // LANES):
        ones_v[pl.ds(k * LANES, LANES)] = jnp.full((LANES,), 1.0, jnp.float32)

    plsc.subcore_barrier()

    def super_body(s, carry):
        c0 = s * SUP
        pltpu.sync_copy(src_hbm.at[wid, pl.ds(c0, SUP)], idx_src)
        pltpu.sync_copy(dst_hbm.at[wid, pl.ds(c0, SUP)], idx_dst)
        pltpu.sync_copy(typ_hbm.at[wid, pl.ds(c0, SUP)], idx_typ)
        n_inner = jnp.minimum(SUP, N_CHUNKS - c0)

        def chunk_body(i, c1):
            src_ids = idx_src.at[i]
            typ_ids = idx_typ.at[i]
            dst_ids = idx_dst.at[i]
            cg = pltpu.async_copy(emb_hbm.at[src_ids], emb_v2.at[0], sem_g)
            cr = pltpu.async_copy(rel_hbm.at[typ_ids], rel_v2.at[0], sem_r)
            cg.wait()
            cr.wait()

            @functools.partial(plsc.parallel_loop, 0, CH, unroll=8)
            def mul_body(r):
                for k in range(D // LANES):
                    sl = pl.ds(k * LANES, LANES)
                    emb_v2[0, r, sl] = emb_v2[0, r, sl] * rel_v2[0, r, sl]

            pltpu.sync_copy(emb_v2.at[0], agg_sh.at[dst_ids], add=True)
            pltpu.sync_copy(ones_v, deg_sh.at[dst_ids], add=True)
            return c1

        lax.fori_loop(0, n_inner, chunk_body, 0)
        return carry

    lax.fori_loop(0, N_SUP, super_body, 0)

    plsc.subcore_barrier()

    @pl.when(sid < NS - 1)
    def _write_agg():
        pltpu.sync_copy(agg_sh.at[pl.ds(row0, ROWS_PER_TILE)],
                        agg_out.at[cid, pl.ds(row0, ROWS_PER_TILE)])

    @pl.when(sid == NS - 1)
    def _write_agg_last():
        pltpu.sync_copy(agg_sh.at[pl.ds((NS - 1) * ROWS_PER_TILE, LAST_TILE_ROWS)],
                        agg_out.at[cid, pl.ds((NS - 1) * ROWS_PER_TILE, LAST_TILE_ROWS)])

    @pl.when(sid == 0)
    def _write_deg():
        pltpu.sync_copy(deg_sh, deg_out.at[cid])


_sc_agg = functools.partial(
    pl.kernel,
    out_type=[
        jax.ShapeDtypeStruct((NC, N_NODES, D), jnp.float32),
        jax.ShapeDtypeStruct((NC, N_NODES), jnp.float32),
    ],
    mesh=plsc.VectorSubcoreMesh(core_axis_name="c", subcore_axis_name="s"),
    scratch_types=[
        pltpu.VMEM((SUP, CH), jnp.int32),
        pltpu.VMEM((SUP, CH), jnp.int32),
        pltpu.VMEM((SUP, CH), jnp.int32),
        pltpu.VMEM((2, CH, D), jnp.float32),
        pltpu.VMEM((2, CH, D), jnp.float32),
        pltpu.VMEM((CH,), jnp.float32),
        pltpu.VMEM_SHARED((N_NODES, D), jnp.float32),
        pltpu.VMEM_SHARED((N_NODES,), jnp.float32),
        pltpu.SemaphoreType.DMA,
        pltpu.SemaphoreType.DMA,
        pltpu.SemaphoreType.DMA,
        pltpu.SemaphoreType.DMA,
    ],
)(_sc_agg_body)


ROW_BLK = 1000


def _tc_finish_body(aggp_ref, degp_ref, emb_ref, w_ref, wl_ref, b_ref, out_ref):
    agg = aggp_ref[0] + aggp_ref[1]
    deg = degp_ref[0, :, 0] + degp_ref[1, :, 0]
    norm = 1.0 / jnp.maximum(deg, 1.0)
    x = agg * norm[:, None]
    acc = jnp.dot(x, w_ref[...], preferred_element_type=jnp.float32)
    acc = acc + jnp.dot(emb_ref[...], wl_ref[...],
                        preferred_element_type=jnp.float32)
    out_ref[...] = jnp.tanh(acc + b_ref[...])


def _tc_finish(aggp, degp, emb, W, W_loop, b2d):
    grid = (N_NODES // ROW_BLK,)
    return pl.pallas_call(
        _tc_finish_body,
        grid=grid,
        in_specs=[
            pl.BlockSpec((NC, ROW_BLK, D), lambda i: (0, i, 0)),
            pl.BlockSpec((NC, ROW_BLK, 1), lambda i: (0, i, 0)),
            pl.BlockSpec((ROW_BLK, D), lambda i: (i, 0)),
            pl.BlockSpec((D, D), lambda i: (0, 0)),
            pl.BlockSpec((D, D), lambda i: (0, 0)),
            pl.BlockSpec((1, D), lambda i: (0, 0)),
        ],
        out_specs=pl.BlockSpec((ROW_BLK, D), lambda i: (i, 0)),
        out_shape=jax.ShapeDtypeStruct((N_NODES, D), jnp.float32),
    )(aggp, degp.reshape(NC, N_NODES, 1), emb, W, W_loop, b2d)


def _slab(x):
    x = x.reshape(NW, E_PER_TILE)
    x = jnp.pad(x, ((0, 0), (0, SLAB_CHUNKS * CH - E_PER_TILE)))
    return x.reshape(NW, SLAB_CHUNKS, CH)


def kernel(t, emb, edge_index, edge_type, W, W_loop, rel_emb, b):
    src = _slab(edge_index[0])
    dst = _slab(edge_index[1])
    typ = _slab(edge_type)
    zagg = jnp.zeros((N_NODES, D), jnp.float32)
    zdeg = jnp.zeros((N_NODES,), jnp.float32)
    aggp, degp = _sc_agg(src, dst, typ, emb, rel_emb, zagg, zdeg)
    return _tc_finish(aggp, degp, emb, W, W_loop, b.reshape(1, D))
